# baseline (speedup 1.0000x reference)
import functools

import jax
import jax.numpy as jnp
from jax import lax
from jax.experimental import pallas as pl
from jax.experimental.pallas import tpu as pltpu
from jax.experimental.pallas import tpu_sc as plsc

N = 10000
E = 160000
D_IN = 256
D_H = 512

NC = 2
NS = 16
NW = NC * NS
ROWS_PER_W = 320
LAST_ROWS = N - (NW - 1) * ROWS_PER_W
CAP = 6144
G = 64
CE = 8000
UNROLL = 4
DUMMY_ROW = ROWS_PER_W
ACC_ROWS = ROWS_PER_W + 8


def _partition_body(edges_hbm, src_out, dst_out, cnt_out, dinv_out,
                    src_stage, dst_stage, src_stage2, dst_stage2,
                    src_list, dst_list, deg, vec16, ssem_a, ssem_b):
    w = lax.axis_index("s") * NC + lax.axis_index("c")
    lo = w * ROWS_PER_W

    one_f = jnp.full((16,), 1.0, dtype=jnp.float32)
    for k in range(ACC_ROWS // 16):
        deg[pl.ds(k * 16, 16)] = one_f

    n_chunks = E // CE
    stages = ((src_stage, dst_stage, ssem_a), (src_stage2, dst_stage2, ssem_b))

    def stage_chunk(c, pair):
        ss, ds_, sem = stages[pair]
        pltpu.async_copy(edges_hbm.at[pl.ds(c * CE, CE)], ss, sem)
        pltpu.async_copy(edges_hbm.at[pl.ds(E + c * CE, CE)], ds_, sem)

    def stage_wait(pair):
        ss, ds_, sem = stages[pair]
        pltpu.make_async_copy(edges_hbm.at[pl.ds(0, CE)], ss, sem).wait()
        pltpu.make_async_copy(edges_hbm.at[pl.ds(0, CE)], ds_, sem).wait()

    stage_chunk(0, 0)
    ptr = jnp.int32(0)
    for c in range(n_chunks):
        pair = c % 2
        stage_wait(pair)
        if c + 1 < n_chunks:
            stage_chunk(c + 1, 1 - pair)
        ss, ds_, _ = stages[pair]

        def group_body(gi, p, ss=ss, ds_=ds_):
            for u in range(UNROLL):
                off = (gi * UNROLL + u) * 16
                sv = ss[pl.ds(off, 16)]
                dv = ds_[pl.ds(off, 16)]
                mask = (dv >= lo) & (dv < lo + ROWS_PER_W)
                dl = dv - lo
                plsc.addupdate_scatter(deg, [dl], one_f, mask=mask)
                plsc.store_compressed(src_list.at[pl.ds(p, 16)], sv, mask=mask)
                plsc.store_compressed(dst_list.at[pl.ds(p, 16)], dl, mask=mask)
                cnt = plsc.all_reduce_population_count(mask)[0]
                p = jnp.minimum(p + cnt, CAP - G - 16)
            return p

        ptr = lax.fori_loop(0, CE // (16 * UNROLL), group_body, ptr)

    zero_i = jnp.zeros((16,), dtype=jnp.int32)
    dummy_i = jnp.full((16,), DUMMY_ROW, dtype=jnp.int32)
    for k in range(G // 16):
        src_list[pl.ds(ptr + k * 16, 16)] = zero_i
        dst_list[pl.ds(ptr + k * 16, 16)] = dummy_i

    nch = (ptr + (G - 1)) // G
    vec16[...] = jnp.broadcast_to(nch, (16,)).astype(jnp.int32)
    pltpu.sync_copy(vec16, cnt_out.at[pl.ds(w * 16, 16)])

    for k in range(ROWS_PER_W // 16):
        deg[pl.ds(k * 16, 16)] = 1.0 / deg[pl.ds(k * 16, 16)]
    pltpu.sync_copy(deg.at[pl.ds(0, ROWS_PER_W)],
                    dinv_out.at[pl.ds(w * ROWS_PER_W, ROWS_PER_W)])

    pltpu.sync_copy(src_list, src_out.at[pl.ds(w * CAP, CAP)])
    pltpu.sync_copy(dst_list, dst_out.at[pl.ds(w * CAP, CAP)])


def _hop_body(h_hbm, src_hbm, dst_hbm, cnt_hbm, dinv_hbm, out_hbm,
              src_all, dst_all, rows0, rows1, dinv_v, cnt_v, acc, sem0, sem1):
    w = lax.axis_index("s") * NC + lax.axis_index("c")
    lo = w * ROWS_PER_W

    @pl.when(w < NW - 1)
    def _():
        pltpu.sync_copy(h_hbm.at[pl.ds(lo, ROWS_PER_W)],
                        acc.at[pl.ds(0, ROWS_PER_W)])

    @pl.when(w == NW - 1)
    def _():
        pltpu.sync_copy(h_hbm.at[pl.ds(lo, LAST_ROWS)],
                        acc.at[pl.ds(0, LAST_ROWS)])

    pltpu.sync_copy(cnt_hbm.at[pl.ds(w * 16, 16)], cnt_v)
    pltpu.sync_copy(dinv_hbm.at[pl.ds(w * ROWS_PER_W, ROWS_PER_W)], dinv_v)
    pltpu.sync_copy(src_hbm.at[pl.ds(w * CAP, CAP)], src_all)
    pltpu.sync_copy(dst_hbm.at[pl.ds(w * CAP, CAP)], dst_all)
    nch = cnt_v[...][0]

    def gather(c, rows, sem):
        pltpu.async_copy(h_hbm.at[src_all.at[pl.ds(c * G, G)]], rows, sem)

    def wait_rows(rows, sem):
        pltpu.make_async_copy(h_hbm.at[pl.ds(0, G)], rows, sem).wait()

    @pl.when(nch > 0)
    def _():
        gather(0, rows0, sem0)

    def chunk_body(c, carry):
        def do(rows_cur, sem_cur, rows_nxt, sem_nxt):
            wait_rows(rows_cur, sem_cur)

            @pl.when(c + 1 < nch)
            def _():
                gather(c + 1, rows_nxt, sem_nxt)

            def group_body(q, carry2):
                dvec = dst_all[pl.ds(c * G + q * 16, 16)]
                e0 = q * 16
                for k in range(16):
                    d = dvec[k]
                    e = e0 + k
                    vals = [rows_cur[e, pl.ds(j * 16, 16)]
                            for j in range(D_IN // 16)]
                    for j in range(D_IN // 16):
                        plsc.addupdate(acc.at[d, pl.ds(j * 16, 16)], vals[j])
                return carry2

            lax.fori_loop(0, G // 16, group_body, jnp.int32(0))

        @pl.when(c % 2 == 0)
        def _():
            do(rows0, sem0, rows1, sem1)

        @pl.when(c % 2 == 1)
        def _():
            do(rows1, sem1, rows0, sem0)

        return carry

    lax.fori_loop(0, nch, chunk_body, jnp.int32(0))

    def scale_body(q, carry):
        svec = dinv_v[pl.ds(q * 16, 16)]
        r0 = q * 16
        for k in range(16):
            s = svec[k]
            r = r0 + k
            vals = [acc[r, pl.ds(j * 16, 16)] * s for j in range(D_IN // 16)]
            for j in range(D_IN // 16):
                acc[r, pl.ds(j * 16, 16)] = vals[j]
        return carry

    lax.fori_loop(0, ROWS_PER_W // 16, scale_body, jnp.int32(0))

    @pl.when(w < NW - 1)
    def _():
        pltpu.sync_copy(acc.at[pl.ds(0, ROWS_PER_W)],
                        out_hbm.at[pl.ds(lo, ROWS_PER_W)])

    @pl.when(w == NW - 1)
    def _():
        pltpu.sync_copy(acc.at[pl.ds(0, LAST_ROWS)],
                        out_hbm.at[pl.ds(lo, LAST_ROWS)])


def _make_sc_kernels():
    mesh = plsc.VectorSubcoreMesh(core_axis_name="c", subcore_axis_name="s")

    partition = pl.kernel(
        _partition_body,
        out_type=(
            jax.ShapeDtypeStruct((NW * CAP,), jnp.int32),
            jax.ShapeDtypeStruct((NW * CAP,), jnp.int32),
            jax.ShapeDtypeStruct((NW * 16,), jnp.int32),
            jax.ShapeDtypeStruct((NW * ROWS_PER_W,), jnp.float32),
        ),
        mesh=mesh,
        compiler_params=pltpu.CompilerParams(needs_layout_passes=False),
        scratch_types=[
            pltpu.VMEM((CE,), jnp.int32),
            pltpu.VMEM((CE,), jnp.int32),
            pltpu.VMEM((CE,), jnp.int32),
            pltpu.VMEM((CE,), jnp.int32),
            pltpu.VMEM((CAP,), jnp.int32),
            pltpu.VMEM((CAP,), jnp.int32),
            pltpu.VMEM((ACC_ROWS,), jnp.float32),
            pltpu.VMEM((16,), jnp.int32),
            pltpu.SemaphoreType.DMA,
            pltpu.SemaphoreType.DMA,
        ],
    )

    hop = pl.kernel(
        _hop_body,
        out_type=jax.ShapeDtypeStruct((N, D_IN), jnp.float32),
        mesh=mesh,
        compiler_params=pltpu.CompilerParams(needs_layout_passes=False),
        scratch_types=[
            pltpu.VMEM((CAP,), jnp.int32),
            pltpu.VMEM((CAP,), jnp.int32),
            pltpu.VMEM((G, D_IN), jnp.float32),
            pltpu.VMEM((G, D_IN), jnp.float32),
            pltpu.VMEM((ROWS_PER_W,), jnp.float32),
            pltpu.VMEM((16,), jnp.int32),
            pltpu.VMEM((ACC_ROWS, D_IN), jnp.float32),
            pltpu.SemaphoreType.DMA,
            pltpu.SemaphoreType.DMA,
        ],
    )
    return partition, hop


_partition_call, _hop_call = _make_sc_kernels()

BM = 1000


def _dense_body(x0, x1, x2, x3, W_r, b_r, g_r, be_r, Wf_r, bf_r, gf_r, bef_r,
                out_r):
    eps = 1e-5
    xs = (x0, x1, x2, x3)
    s = None
    for i in range(4):
        z = jnp.dot(xs[i][...].astype(jnp.bfloat16),
                    W_r[i].astype(jnp.bfloat16),
                    preferred_element_type=jnp.float32)
        z = z + b_r[i][None, :]
        z = jnp.maximum(z, 0.0)
        mu = jnp.mean(z, axis=-1, keepdims=True)
        var = jnp.mean((z - mu) * (z - mu), axis=-1, keepdims=True)
        z = (z - mu) / jnp.sqrt(var + eps) * g_r[i][None, :] + be_r[i][None, :]
        s = z if s is None else s + z
    o = jnp.dot(s.astype(jnp.bfloat16), Wf_r[...].astype(jnp.bfloat16),
                preferred_element_type=jnp.float32)
    o = o + bf_r[0][None, :]
    o = jnp.maximum(o, 0.0)
    mu = jnp.mean(o, axis=-1, keepdims=True)
    var = jnp.mean((o - mu) * (o - mu), axis=-1, keepdims=True)
    out_r[...] = (o - mu) / jnp.sqrt(var + eps) * gf_r[0][None, :] \
        + bef_r[0][None, :]


@functools.partial(jax.jit, static_argnames=())
def _dense_call(x0, x1, x2, x3, W, b, g, be, Wf, bf, gf, bef):
    row_spec = pl.BlockSpec((BM, D_IN), lambda i: (i, 0))
    return pl.pallas_call(
        _dense_body,
        grid=(N // BM,),
        in_specs=[
            row_spec, row_spec, row_spec, row_spec,
            pl.BlockSpec((4, D_IN, D_H), lambda i: (0, 0, 0)),
            pl.BlockSpec((4, D_H), lambda i: (0, 0)),
            pl.BlockSpec((4, D_H), lambda i: (0, 0)),
            pl.BlockSpec((4, D_H), lambda i: (0, 0)),
            pl.BlockSpec((D_H, D_H), lambda i: (0, 0)),
            pl.BlockSpec((1, D_H), lambda i: (0, 0)),
            pl.BlockSpec((1, D_H), lambda i: (0, 0)),
            pl.BlockSpec((1, D_H), lambda i: (0, 0)),
        ],
        out_specs=pl.BlockSpec((BM, D_H), lambda i: (i, 0)),
        out_shape=jax.ShapeDtypeStruct((N, D_H), jnp.float32),
    )(x0, x1, x2, x3, W, b, g, be, Wf, bf, gf, bef)


def kernel(x, edge_index, W, b, g, be, Wf, bf, gf, bef):
    src_l, dst_l, cnt, dinv = _partition_call(edge_index.reshape(-1))
    h1 = _hop_call(x, src_l, dst_l, cnt, dinv)
    h2 = _hop_call(h1, src_l, dst_l, cnt, dinv)
    h3 = _hop_call(h2, src_l, dst_l, cnt, dinv)
    return _dense_call(x, h1, h2, h3, W, b, g, be,
                       Wf, bf.reshape(1, D_H), gf.reshape(1, D_H),
                       bef.reshape(1, D_H))


# speedup vs baseline: 1.0111x; 1.0111x over previous
import functools

import jax
import jax.numpy as jnp
from jax import lax
from jax.experimental import pallas as pl
from jax.experimental.pallas import tpu as pltpu
from jax.experimental.pallas import tpu_sc as plsc

N = 10000
E = 160000
D_IN = 256
D_H = 512

NC = 2
NS = 16
NW = NC * NS
ROWS_PER_W = 320
LAST_ROWS = N - (NW - 1) * ROWS_PER_W
CAP = 6144
G = 64
CE = 8000
UNROLL = 4
DUMMY_ROW = ROWS_PER_W
ACC_ROWS = ROWS_PER_W + 8


def _partition_body(edges_hbm, src_out, dst_out, cnt_out, dinv_out,
                    src_stage, dst_stage, src_stage2, dst_stage2,
                    src_list, dst_list, deg, vec16, ssem_a, ssem_b):
    w = lax.axis_index("s") * NC + lax.axis_index("c")
    lo = w * ROWS_PER_W

    one_f = jnp.full((16,), 1.0, dtype=jnp.float32)
    for k in range(ACC_ROWS // 16):
        deg[pl.ds(k * 16, 16)] = one_f

    n_chunks = E // CE
    stages = ((src_stage, dst_stage, ssem_a), (src_stage2, dst_stage2, ssem_b))

    def stage_chunk(c, pair):
        ss, ds_, sem = stages[pair]
        pltpu.async_copy(edges_hbm.at[pl.ds(c * CE, CE)], ss, sem)
        pltpu.async_copy(edges_hbm.at[pl.ds(E + c * CE, CE)], ds_, sem)

    def stage_wait(pair):
        ss, ds_, sem = stages[pair]
        pltpu.make_async_copy(edges_hbm.at[pl.ds(0, CE)], ss, sem).wait()
        pltpu.make_async_copy(edges_hbm.at[pl.ds(0, CE)], ds_, sem).wait()

    stage_chunk(0, 0)
    ptr = jnp.int32(0)
    for c in range(n_chunks):
        pair = c % 2
        stage_wait(pair)
        if c + 1 < n_chunks:
            stage_chunk(c + 1, 1 - pair)
        ss, ds_, _ = stages[pair]

        def group_body(gi, p, ss=ss, ds_=ds_):
            for u in range(UNROLL):
                off = (gi * UNROLL + u) * 16
                sv = ss[pl.ds(off, 16)]
                dv = ds_[pl.ds(off, 16)]
                mask = (dv >= lo) & (dv < lo + ROWS_PER_W)
                dl = dv - lo
                plsc.addupdate_scatter(deg, [dl], one_f, mask=mask)
                plsc.store_compressed(src_list.at[pl.ds(p, 16)], sv, mask=mask)
                plsc.store_compressed(dst_list.at[pl.ds(p, 16)], dl, mask=mask)
                cnt = plsc.all_reduce_population_count(mask)[0]
                p = jnp.minimum(p + cnt, CAP - G - 16)
            return p

        ptr = lax.fori_loop(0, CE // (16 * UNROLL), group_body, ptr)

    zero_i = jnp.zeros((16,), dtype=jnp.int32)
    dummy_i = jnp.full((16,), DUMMY_ROW, dtype=jnp.int32)
    for k in range(G // 16):
        src_list[pl.ds(ptr + k * 16, 16)] = zero_i
        dst_list[pl.ds(ptr + k * 16, 16)] = dummy_i

    nch = (ptr + (G - 1)) // G
    vec16[...] = jnp.broadcast_to(nch, (16,)).astype(jnp.int32)
    pltpu.sync_copy(vec16, cnt_out.at[pl.ds(w * 16, 16)])

    for k in range(ROWS_PER_W // 16):
        deg[pl.ds(k * 16, 16)] = 1.0 / deg[pl.ds(k * 16, 16)]
    pltpu.sync_copy(deg.at[pl.ds(0, ROWS_PER_W)],
                    dinv_out.at[pl.ds(w * ROWS_PER_W, ROWS_PER_W)])

    pltpu.sync_copy(src_list, src_out.at[pl.ds(w * CAP, CAP)])
    pltpu.sync_copy(dst_list, dst_out.at[pl.ds(w * CAP, CAP)])


def _hop_body(h_hbm, src_hbm, dst_hbm, cnt_hbm, dinv_hbm, out_hbm,
              src_all, dst_all, rows0, rows1, dinv_v, cnt_v, acc, sem0, sem1):
    w = lax.axis_index("s") * NC + lax.axis_index("c")
    lo = w * ROWS_PER_W

    pltpu.async_copy(cnt_hbm.at[pl.ds(w * 16, 16)], cnt_v, sem0)
    pltpu.async_copy(dinv_hbm.at[pl.ds(w * ROWS_PER_W, ROWS_PER_W)],
                     dinv_v, sem0)
    pltpu.async_copy(src_hbm.at[pl.ds(w * CAP, CAP)], src_all, sem0)
    pltpu.async_copy(dst_hbm.at[pl.ds(w * CAP, CAP)], dst_all, sem0)
    pltpu.make_async_copy(cnt_hbm.at[pl.ds(0, 16)], cnt_v, sem0).wait()
    pltpu.make_async_copy(dinv_hbm.at[pl.ds(0, ROWS_PER_W)], dinv_v,
                          sem0).wait()
    pltpu.make_async_copy(src_hbm.at[pl.ds(0, CAP)], src_all, sem0).wait()
    pltpu.make_async_copy(dst_hbm.at[pl.ds(0, CAP)], dst_all, sem0).wait()
    nch = cnt_v[...][0]

    def gather(c, rows, sem):
        pltpu.async_copy(h_hbm.at[src_all.at[pl.ds(c * G, G)]], rows, sem)

    def wait_rows(rows, sem):
        pltpu.make_async_copy(h_hbm.at[pl.ds(0, G)], rows, sem).wait()

    @pl.when(nch > 0)
    def _():
        gather(0, rows0, sem0)

    @pl.when(w < NW - 1)
    def _():
        pltpu.sync_copy(h_hbm.at[pl.ds(lo, ROWS_PER_W)],
                        acc.at[pl.ds(0, ROWS_PER_W)])

    @pl.when(w == NW - 1)
    def _():
        pltpu.sync_copy(h_hbm.at[pl.ds(lo, LAST_ROWS)],
                        acc.at[pl.ds(0, LAST_ROWS)])

    def chunk_body(c, carry):
        def do(rows_cur, sem_cur, rows_nxt, sem_nxt):
            wait_rows(rows_cur, sem_cur)

            @pl.when(c + 1 < nch)
            def _():
                gather(c + 1, rows_nxt, sem_nxt)

            def group_body(q, carry2):
                dvec = dst_all[pl.ds(c * G + q * 16, 16)]
                e0 = q * 16
                for k in range(16):
                    d = dvec[k]
                    e = e0 + k
                    vals = [rows_cur[e, pl.ds(j * 16, 16)]
                            for j in range(D_IN // 16)]
                    for j in range(D_IN // 16):
                        plsc.addupdate(acc.at[d, pl.ds(j * 16, 16)], vals[j])
                return carry2

            lax.fori_loop(0, G // 16, group_body, jnp.int32(0))

        @pl.when(c % 2 == 0)
        def _():
            do(rows0, sem0, rows1, sem1)

        @pl.when(c % 2 == 1)
        def _():
            do(rows1, sem1, rows0, sem0)

        return carry

    lax.fori_loop(0, nch, chunk_body, jnp.int32(0))

    def scale_body(q, carry):
        svec = dinv_v[pl.ds(q * 16, 16)]
        r0 = q * 16
        for k in range(16):
            s = svec[k]
            r = r0 + k
            vals = [acc[r, pl.ds(j * 16, 16)] * s for j in range(D_IN // 16)]
            for j in range(D_IN // 16):
                acc[r, pl.ds(j * 16, 16)] = vals[j]
        return carry

    lax.fori_loop(0, ROWS_PER_W // 16, scale_body, jnp.int32(0))

    @pl.when(w < NW - 1)
    def _():
        pltpu.sync_copy(acc.at[pl.ds(0, ROWS_PER_W)],
                        out_hbm.at[pl.ds(lo, ROWS_PER_W)])

    @pl.when(w == NW - 1)
    def _():
        pltpu.sync_copy(acc.at[pl.ds(0, LAST_ROWS)],
                        out_hbm.at[pl.ds(lo, LAST_ROWS)])


def _make_sc_kernels():
    mesh = plsc.VectorSubcoreMesh(core_axis_name="c", subcore_axis_name="s")

    partition = pl.kernel(
        _partition_body,
        out_type=(
            jax.ShapeDtypeStruct((NW * CAP,), jnp.int32),
            jax.ShapeDtypeStruct((NW * CAP,), jnp.int32),
            jax.ShapeDtypeStruct((NW * 16,), jnp.int32),
            jax.ShapeDtypeStruct((NW * ROWS_PER_W,), jnp.float32),
        ),
        mesh=mesh,
        compiler_params=pltpu.CompilerParams(needs_layout_passes=False),
        scratch_types=[
            pltpu.VMEM((CE,), jnp.int32),
            pltpu.VMEM((CE,), jnp.int32),
            pltpu.VMEM((CE,), jnp.int32),
            pltpu.VMEM((CE,), jnp.int32),
            pltpu.VMEM((CAP,), jnp.int32),
            pltpu.VMEM((CAP,), jnp.int32),
            pltpu.VMEM((ACC_ROWS,), jnp.float32),
            pltpu.VMEM((16,), jnp.int32),
            pltpu.SemaphoreType.DMA,
            pltpu.SemaphoreType.DMA,
        ],
    )

    hop = pl.kernel(
        _hop_body,
        out_type=jax.ShapeDtypeStruct((N, D_IN), jnp.float32),
        mesh=mesh,
        compiler_params=pltpu.CompilerParams(needs_layout_passes=False),
        scratch_types=[
            pltpu.VMEM((CAP,), jnp.int32),
            pltpu.VMEM((CAP,), jnp.int32),
            pltpu.VMEM((G, D_IN), jnp.float32),
            pltpu.VMEM((G, D_IN), jnp.float32),
            pltpu.VMEM((ROWS_PER_W,), jnp.float32),
            pltpu.VMEM((16,), jnp.int32),
            pltpu.VMEM((ACC_ROWS, D_IN), jnp.float32),
            pltpu.SemaphoreType.DMA,
            pltpu.SemaphoreType.DMA,
        ],
    )
    return partition, hop


_partition_call, _hop_call = _make_sc_kernels()

BM = 1000


def _dense_body(x0, x1, x2, x3, W_r, b_r, g_r, be_r, Wf_r, bf_r, gf_r, bef_r,
                out_r):
    eps = 1e-5
    xs = (x0, x1, x2, x3)
    s = None
    for i in range(4):
        z = jnp.dot(xs[i][...], W_r[i], preferred_element_type=jnp.float32)
        z = z + b_r[i][None, :]
        z = jnp.maximum(z, 0.0)
        mu = jnp.mean(z, axis=-1, keepdims=True)
        var = jnp.mean((z - mu) * (z - mu), axis=-1, keepdims=True)
        z = (z - mu) / jnp.sqrt(var + eps) * g_r[i][None, :] + be_r[i][None, :]
        s = z if s is None else s + z
    o = jnp.dot(s, Wf_r[...], preferred_element_type=jnp.float32)
    o = o + bf_r[0][None, :]
    o = jnp.maximum(o, 0.0)
    mu = jnp.mean(o, axis=-1, keepdims=True)
    var = jnp.mean((o - mu) * (o - mu), axis=-1, keepdims=True)
    out_r[...] = (o - mu) / jnp.sqrt(var + eps) * gf_r[0][None, :] \
        + bef_r[0][None, :]


@functools.partial(jax.jit, static_argnames=())
def _dense_call(x0, x1, x2, x3, W, b, g, be, Wf, bf, gf, bef):
    row_spec = pl.BlockSpec((BM, D_IN), lambda i: (i, 0))
    return pl.pallas_call(
        _dense_body,
        grid=(N // BM,),
        in_specs=[
            row_spec, row_spec, row_spec, row_spec,
            pl.BlockSpec((4, D_IN, D_H), lambda i: (0, 0, 0)),
            pl.BlockSpec((4, D_H), lambda i: (0, 0)),
            pl.BlockSpec((4, D_H), lambda i: (0, 0)),
            pl.BlockSpec((4, D_H), lambda i: (0, 0)),
            pl.BlockSpec((D_H, D_H), lambda i: (0, 0)),
            pl.BlockSpec((1, D_H), lambda i: (0, 0)),
            pl.BlockSpec((1, D_H), lambda i: (0, 0)),
            pl.BlockSpec((1, D_H), lambda i: (0, 0)),
        ],
        out_specs=pl.BlockSpec((BM, D_H), lambda i: (i, 0)),
        out_shape=jax.ShapeDtypeStruct((N, D_H), jnp.float32),
    )(x0, x1, x2, x3, W, b, g, be, Wf, bf, gf, bef)


def kernel(x, edge_index, W, b, g, be, Wf, bf, gf, bef):
    src_l, dst_l, cnt, dinv = _partition_call(edge_index.reshape(-1))
    h1 = _hop_call(x, src_l, dst_l, cnt, dinv)
    h2 = _hop_call(h1, src_l, dst_l, cnt, dinv)
    h3 = _hop_call(h2, src_l, dst_l, cnt, dinv)
    return _dense_call(x, h1, h2, h3, W, b, g, be,
                       Wf, bf.reshape(1, D_H), gf.reshape(1, D_H),
                       bef.reshape(1, D_H))
